# Initial kernel scaffold; baseline (speedup 1.0000x reference)
import jax
import jax.numpy as jnp
from jax.experimental import pallas as pl


def kernel(seq, pssm, token_table, pos_table):
    raise NotImplementedError("write your pallas kernel here")


# speedup vs baseline: 4.0860x; 4.0860x over previous
import jax
import jax.numpy as jnp
from jax.experimental import pallas as pl

B = 1024
L = 1024
VOCAB = 21
SEQ_EMB = 44
POS_EMB = 64
BB = 16


def _tc_kernel(seq_ref, pssm_ref, tok_ref, pos_ref, out_ref):
    seq = seq_ref[...]
    vocab_iota = jax.lax.broadcasted_iota(jnp.int32, (1, 1, VOCAB), 2)
    onehot = (seq[..., None] == vocab_iota).astype(jnp.float32)
    emb = jax.lax.dot_general(
        onehot.reshape(BB * L, VOCAB),
        tok_ref[...],
        (((1,), (0,)), ((), ())),
        preferred_element_type=jnp.float32,
    ).reshape(BB, L, SEQ_EMB)
    x = jnp.concatenate([emb, pssm_ref[...]], axis=-1)
    out_ref[...] = x + pos_ref[...][None]


def kernel(seq, pssm, token_table, pos_table):
    seq = seq.astype(jnp.int32)
    grid = (B // BB,)
    return pl.pallas_call(
        _tc_kernel,
        grid=grid,
        in_specs=[
            pl.BlockSpec((BB, L), lambda i: (i, 0)),
            pl.BlockSpec((BB, L, POS_EMB - SEQ_EMB), lambda i: (i, 0, 0)),
            pl.BlockSpec((VOCAB, SEQ_EMB), lambda i: (0, 0)),
            pl.BlockSpec((L, POS_EMB), lambda i: (0, 0)),
        ],
        out_specs=pl.BlockSpec((BB, L, POS_EMB), lambda i: (i, 0, 0)),
        out_shape=jax.ShapeDtypeStruct((B, L, POS_EMB), jnp.float32),
    )(seq, pssm, token_table, pos_table)
